# baseline (speedup 1.0000x reference)
import functools

import jax
import jax.numpy as jnp
import numpy as np
from jax.experimental import pallas as pl

N_NODES = 10000
N_EDGES = 320000
HIDDEN = 128
HEADS = 16
HEAD_DIM = HIDDEN // HEADS

_NODE_BLK = 1000
_EDGE_BLK = 8000


def _qkv_body(x_ref, wq_ref, bq_ref, wk_ref, bk_ref, wv_ref, bv_ref,
              q_ref, k_ref, v_ref):
    xb = x_ref[...]
    q_ref[...] = jnp.dot(xb, wq_ref[...],
                         preferred_element_type=jnp.float32) + bq_ref[...]
    k_ref[...] = jnp.dot(xb, wk_ref[...],
                         preferred_element_type=jnp.float32) + bk_ref[...]
    v_ref[...] = jnp.dot(xb, wv_ref[...],
                         preferred_element_type=jnp.float32) + bv_ref[...]


def _qkv(x, wq, bq, wk, bk, wv, bv):
    nblk = N_NODES // _NODE_BLK
    wspec = pl.BlockSpec((HIDDEN, HIDDEN), lambda i: (0, 0))
    bspec = pl.BlockSpec((1, HIDDEN), lambda i: (0, 0))
    xspec = pl.BlockSpec((_NODE_BLK, HIDDEN), lambda i: (i, 0))
    out = pl.pallas_call(
        _qkv_body,
        grid=(nblk,),
        in_specs=[xspec, wspec, bspec, wspec, bspec, wspec, bspec],
        out_specs=[xspec, xspec, xspec],
        out_shape=[jax.ShapeDtypeStruct((N_NODES, HIDDEN), jnp.float32)] * 3,
    )(x, wq, bq.reshape(1, -1), wk, bk.reshape(1, -1), wv, bv.reshape(1, -1))
    return out


def _edge_proj_body(ea_ref, we_ref, e_ref):
    e_ref[...] = jnp.dot(ea_ref[...], we_ref[...],
                         preferred_element_type=jnp.float32)


def _edge_proj(edge_attr, we):
    eblk = N_EDGES // _EDGE_BLK
    return pl.pallas_call(
        _edge_proj_body,
        grid=(eblk,),
        in_specs=[pl.BlockSpec((_EDGE_BLK, we.shape[0]), lambda i: (i, 0)),
                  pl.BlockSpec(we.shape, lambda i: (0, 0))],
        out_specs=pl.BlockSpec((_EDGE_BLK, HIDDEN), lambda i: (i, 0)),
        out_shape=jax.ShapeDtypeStruct((N_EDGES, HIDDEN), jnp.float32),
    )(edge_attr, we)


def _post_body(x_ref, agg_ref, ws_ref, bs_ref, w1_ref, b1_ref, w2_ref, b2_ref,
               g_ref, be_ref, y_ref):
    xb = x_ref[...]
    out = agg_ref[...] + jnp.dot(xb, ws_ref[...],
                                 preferred_element_type=jnp.float32) + bs_ref[...]
    h = jnp.dot(out, w1_ref[...], preferred_element_type=jnp.float32) + b1_ref[...]
    h = h * jax.nn.sigmoid(h)
    h = jnp.dot(h, w2_ref[...], preferred_element_type=jnp.float32) + b2_ref[...]
    mu = jnp.mean(h, axis=-1, keepdims=True)
    d = h - mu
    var = jnp.mean(d * d, axis=-1, keepdims=True)
    h = d * jax.lax.rsqrt(var + 1e-5) * g_ref[...] + be_ref[...]
    y_ref[...] = xb + h


def _post(x, agg, p):
    nblk = N_NODES // _NODE_BLK
    wspec = pl.BlockSpec((HIDDEN, HIDDEN), lambda i: (0, 0))
    bspec = pl.BlockSpec((1, HIDDEN), lambda i: (0, 0))
    xspec = pl.BlockSpec((_NODE_BLK, HIDDEN), lambda i: (i, 0))
    return pl.pallas_call(
        _post_body,
        grid=(nblk,),
        in_specs=[xspec, xspec, wspec, bspec, wspec, bspec, wspec, bspec,
                  bspec, bspec],
        out_specs=xspec,
        out_shape=jax.ShapeDtypeStruct((N_NODES, HIDDEN), jnp.float32),
    )(x, agg, p['Wskip'], p['bskip'].reshape(1, -1), p['W1'],
      p['b1'].reshape(1, -1), p['W2'], p['b2'].reshape(1, -1),
      p['ln_g'].reshape(1, -1), p['ln_b'].reshape(1, -1))


def _edge_phase(q, k, v, e, src, dst):
    qh = q.reshape(-1, HEADS, HEAD_DIM)
    kh = k.reshape(-1, HEADS, HEAD_DIM)
    vh = v.reshape(-1, HEADS, HEAD_DIM)
    eh = e.reshape(-1, HEADS, HEAD_DIM)
    q_i = qh[dst]
    k_j = kh[src] + eh
    v_j = vh[src]
    alpha = jnp.sum(q_i * k_j, axis=-1) / np.sqrt(HEAD_DIM)
    amax = jax.ops.segment_max(alpha, dst, num_segments=N_NODES)
    amax = jnp.where(jnp.isfinite(amax), amax, 0.0)
    ex = jnp.exp(alpha - amax[dst])
    den = jax.ops.segment_sum(ex, dst, num_segments=N_NODES) + 1e-16
    alpha = ex / den[dst]
    msg = (v_j + eh) * alpha[:, :, None]
    agg = jax.ops.segment_sum(msg, dst, num_segments=N_NODES)
    return agg.reshape(-1, HIDDEN)


def kernel(x, edge_index, edge_attr, params):
    src, dst = edge_index[0], edge_index[1]
    for p in params:
        q, k, v = _qkv(x, p['Wq'], p['bq'], p['Wk'], p['bk'], p['Wv'], p['bv'])
        e = _edge_proj(edge_attr, p['We'])
        agg = _edge_phase(q, k, v, e, src, dst)
        x = _post(x, agg, p)
    return x
